# baseline (speedup 1.0000x reference)
import functools

import jax
import jax.numpy as jnp
from jax import lax
from jax.experimental import pallas as pl
from jax.experimental.pallas import tpu as pltpu
from jax.experimental.pallas import tpu_sc as plsc

_M = 4096
_N = 8192
_NW = 32
_ROWS_PER_W = _M // _NW
_L = 16
_NSLICE = _N // _L


def _sc_kernel(x_hbm, o_hbm, row_v, out_v):
    wid = lax.axis_index("s") * 2 + lax.axis_index("c")
    base = wid * _ROWS_PER_W

    def row_body(r, _):
        row = base + r
        pltpu.sync_copy(x_hbm.at[row], row_v)

        def slice_body(i, carry):
            v = row_v[pl.ds(i * _L, _L)]
            out_v[pl.ds(i * _L, _L)] = jnp.cumsum(v) + carry
            return carry + jnp.sum(v)

        lax.fori_loop(0, _NSLICE, slice_body, jnp.float32(0.0))
        pltpu.sync_copy(out_v, o_hbm.at[row])
        return _

    lax.fori_loop(0, _ROWS_PER_W, row_body, jnp.int32(0))


def kernel(x):
    mesh = plsc.VectorSubcoreMesh(core_axis_name="c", subcore_axis_name="s")
    run = functools.partial(
        pl.kernel,
        mesh=mesh,
        out_type=jax.ShapeDtypeStruct((_M, _N), jnp.float32),
        scratch_types=[
            pltpu.VMEM((_N,), jnp.float32),
            pltpu.VMEM((_N,), jnp.float32),
        ],
        compiler_params=pltpu.CompilerParams(needs_layout_passes=False),
    )(_sc_kernel)
    return run(x)


# speedup vs baseline: 4.4197x; 4.4197x over previous
import functools

import jax
import jax.numpy as jnp
import numpy as np
from jax.experimental import pallas as pl
from jax.experimental.pallas import tpu as pltpu

_N = 8192
_CHUNK = 256
_NCHUNK = _N // _CHUNK
_BLOCK_ROWS = 256


def _split(v):
    hi = v.astype(jnp.bfloat16)
    lo = (v - hi.astype(jnp.float32)).astype(jnp.bfloat16)
    return hi, lo


def _scan_kernel(x_ref, tri_ref, bd_ref, stri_ref, e_ref, o_ref):
    xh = x_ref[...].astype(jnp.bfloat16)
    tri = tri_ref[...]
    bd = bd_ref[...]

    totals = jnp.dot(xh, bd, preferred_element_type=jnp.float32)

    th, tl = _split(totals)
    stri = stri_ref[...]
    carries = jnp.dot(th, stri, preferred_element_type=jnp.float32) + jnp.dot(
        tl, stri, preferred_element_type=jnp.float32
    )
    ch, cl = _split(carries)

    for c in range(_NCHUNK):
        s = slice(c * _CHUNK, (c + 1) * _CHUNK)
        ec = e_ref[:, s]
        o_ref[:, s] = (
            jnp.dot(xh[:, s], tri, preferred_element_type=jnp.float32)
            + jnp.dot(ch, ec, preferred_element_type=jnp.float32)
            + jnp.dot(cl, ec, preferred_element_type=jnp.float32)
        )


@functools.partial(jax.jit, static_argnums=())
def _run(x, tri, bd, stri, e):
    m, n = x.shape
    grid = (m // _BLOCK_ROWS,)
    const = lambda shape: pl.BlockSpec(shape, lambda i: (0, 0))
    return pl.pallas_call(
        _scan_kernel,
        grid=grid,
        in_specs=[
            pl.BlockSpec((_BLOCK_ROWS, n), lambda i: (i, 0)),
            const((_CHUNK, _CHUNK)),
            const((_N, _NCHUNK)),
            const((_NCHUNK, _NCHUNK)),
            const((_NCHUNK, _N)),
        ],
        out_specs=pl.BlockSpec((_BLOCK_ROWS, n), lambda i: (i, 0)),
        out_shape=jax.ShapeDtypeStruct((m, n), x.dtype),
        compiler_params=pltpu.CompilerParams(
            dimension_semantics=("parallel",),
        ),
    )(x, tri, bd, stri, e)


def kernel(x):
    ii, jj = np.indices((_CHUNK, _CHUNK))
    tri = jnp.asarray((ii <= jj), dtype=jnp.bfloat16)
    bi, bj = np.indices((_N, _NCHUNK))
    bd = jnp.asarray((bi // _CHUNK == bj), dtype=jnp.bfloat16)
    ci, cj = np.indices((_NCHUNK, _NCHUNK))
    stri = jnp.asarray((ci < cj), dtype=jnp.bfloat16)
    ri, rj = np.indices((_NCHUNK, _N))
    e = jnp.asarray((rj // _CHUNK == ri), dtype=jnp.bfloat16)
    return _run(x, tri, bd, stri, e)


# speedup vs baseline: 4.9437x; 1.1186x over previous
import functools

import jax
import jax.numpy as jnp
import numpy as np
from jax.experimental import pallas as pl
from jax.experimental.pallas import tpu as pltpu

_N = 8192
_CHUNK = 256
_NCHUNK = _N // _CHUNK
_BLOCK_ROWS = 256


def _split(v):
    hi = v.astype(jnp.bfloat16)
    lo = (v - hi.astype(jnp.float32)).astype(jnp.bfloat16)
    return hi, lo


def _scan_kernel(x_ref, tri_ref, bd_ref, stri_ref, o_ref):
    xh = x_ref[...].astype(jnp.bfloat16)
    tri = tri_ref[...]
    bd = bd_ref[...]

    totals = jnp.dot(xh, bd, preferred_element_type=jnp.float32)

    th, tl = _split(totals)
    stri = stri_ref[...]
    carries = jnp.dot(th, stri, preferred_element_type=jnp.float32) + jnp.dot(
        tl, stri, preferred_element_type=jnp.float32
    )
    for c in range(_NCHUNK):
        s = slice(c * _CHUNK, (c + 1) * _CHUNK)
        bc = jnp.broadcast_to(carries[:, c : c + 1], (carries.shape[0], _CHUNK))
        o_ref[:, s] = (
            jnp.dot(xh[:, s], tri, preferred_element_type=jnp.float32) + bc
        )


@functools.partial(jax.jit, static_argnums=())
def _run(x, tri, bd, stri):
    m, n = x.shape
    grid = (m // _BLOCK_ROWS,)
    const = lambda shape: pl.BlockSpec(shape, lambda i: (0, 0))
    return pl.pallas_call(
        _scan_kernel,
        grid=grid,
        in_specs=[
            pl.BlockSpec((_BLOCK_ROWS, n), lambda i: (i, 0)),
            const((_CHUNK, _CHUNK)),
            const((_N, _NCHUNK)),
            const((_NCHUNK, _NCHUNK)),
        ],
        out_specs=pl.BlockSpec((_BLOCK_ROWS, n), lambda i: (i, 0)),
        out_shape=jax.ShapeDtypeStruct((m, n), x.dtype),
        compiler_params=pltpu.CompilerParams(
            dimension_semantics=("parallel",),
        ),
    )(x, tri, bd, stri)


def kernel(x):
    ii, jj = np.indices((_CHUNK, _CHUNK))
    tri = jnp.asarray((ii <= jj), dtype=jnp.bfloat16)
    bi, bj = np.indices((_N, _NCHUNK))
    bd = jnp.asarray((bi // _CHUNK == bj), dtype=jnp.bfloat16)
    ci, cj = np.indices((_NCHUNK, _NCHUNK))
    stri = jnp.asarray((ci < cj), dtype=jnp.bfloat16)
    return _run(x, tri, bd, stri)
